# Initial kernel scaffold; baseline (speedup 1.0000x reference)
import jax
import jax.numpy as jnp
from jax.experimental import pallas as pl


def kernel(x, edge_index, params):
    raise NotImplementedError("write your pallas kernel here")


# speedup vs baseline: 38.6198x; 38.6198x over previous
import functools
import math

import jax
import jax.numpy as jnp
from jax import lax
from jax.experimental import pallas as pl
from jax.experimental.pallas import tpu as pltpu
from jax.experimental.pallas import tpu_sc as plsc

H = 8
C = 16
HID = 128
NC = 2
NS = 16
NW = NC * NS
T = 128
ACC_W = 144


def _proj_body(x_ref, w_ref, b_ref, q_ref, k_ref, v_ref, s_ref):
    acc = jnp.dot(x_ref[...], w_ref[...], preferred_element_type=jnp.float32)
    acc = acc + b_ref[...]
    q_ref[...] = acc[:, 0:128]
    k_ref[...] = acc[:, 128:256]
    v_ref[...] = acc[:, 256:384]
    s_ref[...] = acc[:, 384:512]


@functools.lru_cache(maxsize=None)
def _make_project(N):
    BLK = 2000
    grid = N // BLK
    return pl.pallas_call(
        _proj_body,
        grid=(grid,),
        in_specs=[
            pl.BlockSpec((BLK, HID), lambda i: (i, 0)),
            pl.BlockSpec((HID, 4 * HID), lambda i: (0, 0)),
            pl.BlockSpec((1, 4 * HID), lambda i: (0, 0)),
        ],
        out_specs=[pl.BlockSpec((BLK, HID), lambda i: (i, 0))] * 4,
        out_shape=[jax.ShapeDtypeStruct((N, HID), jnp.float32)] * 4,
    )


@functools.lru_cache(maxsize=None)
def _make_sc(N, E_pad, E_real):
    EPW = E_pad // NW
    NCH = EPW // T
    NCHG = E_pad // T
    RPT = N // NS
    mesh = plsc.VectorSubcoreMesh(core_axis_name="c", subcore_axis_name="s")

    @functools.partial(
        pl.kernel,
        out_type=(
            jax.ShapeDtypeStruct((NCHG, T, 16), jnp.float32),
            jax.ShapeDtypeStruct((NW, 16), jnp.float32),
        ),
        mesh=mesh,
        compiler_params=pltpu.CompilerParams(needs_layout_passes=False, use_tc_tiling_on_sc=False),
        scratch_types=[
            pltpu.VMEM((T,), jnp.int32),
            pltpu.VMEM((T,), jnp.int32),
            pltpu.VMEM((T, HID), jnp.float32),
            pltpu.VMEM((T, HID), jnp.float32),
            pltpu.VMEM((T, 16), jnp.float32),
            pltpu.VMEM((16,), jnp.float32),
            pltpu.SemaphoreType.DMA,
            pltpu.SemaphoreType.DMA,
        ],
    )
    def pass1(dst_hbm, src_hbm, q_hbm, k_hbm, alpha_hbm, mx_hbm,
              dst_i, src_i, qbuf, kbuf, abuf, mbuf, sem0, sem1):
        wid = lax.axis_index("s") * NC + lax.axis_index("c")
        il = lax.iota(jnp.int32, 16)

        def chunk(ci, mv):
            e0 = wid * EPW + ci * T
            pltpu.sync_copy(dst_hbm.at[pl.ds(e0, T)], dst_i)
            pltpu.sync_copy(src_hbm.at[pl.ds(e0, T)], src_i)
            cq = pltpu.async_copy(q_hbm.at[dst_i], qbuf, sem0)
            ck = pltpu.async_copy(k_hbm.at[src_i], kbuf, sem1)
            cq.wait()
            ck.wait()

            def edge(e, mv):
                row = jnp.zeros((16,), jnp.float32)
                for h in range(H):
                    qv = qbuf[e, pl.ds(h * C, 16)]
                    kv = kbuf[e, pl.ds(h * C, 16)]
                    s = jnp.sum(qv * kv) * (1.0 / math.sqrt(C))
                    row = row + jnp.where(il == h, s, 0.0)
                abuf[e] = row
                return jnp.maximum(mv, row)

            mv = lax.fori_loop(0, T, edge, mv)
            pltpu.sync_copy(abuf, alpha_hbm.at[wid * NCH + ci])
            return mv

        mv = lax.fori_loop(0, NCH, chunk, jnp.full((16,), -1e30, jnp.float32))
        mbuf[...] = mv
        pltpu.sync_copy(mbuf, mx_hbm.at[wid])

    @functools.partial(
        pl.kernel,
        out_type=jax.ShapeDtypeStruct((NC, N, ACC_W), jnp.float32),
        mesh=mesh,
        compiler_params=pltpu.CompilerParams(needs_layout_passes=False, use_tc_tiling_on_sc=False),
        scratch_types=[
            pltpu.VMEM((T,), jnp.int32),
            pltpu.VMEM((T,), jnp.int32),
            pltpu.VMEM((T, HID), jnp.float32),
            pltpu.VMEM((T, 16), jnp.float32),
            pltpu.VMEM((T, ACC_W), jnp.float32),
            pltpu.VMEM((NW, 16), jnp.float32),
            pltpu.VMEM_SHARED((N, ACC_W), jnp.float32),
            pltpu.SemaphoreType.DMA,
        ],
    )
    def pass2(dst_hbm, src_hbm, v_hbm, alpha_hbm, mx_hbm, zeros_hbm, out_hbm,
              dst_i, src_i, vbuf, exbuf, wvbuf, mxbuf, acc, sem0):
        cid = lax.axis_index("c")
        sid = lax.axis_index("s")
        wid = sid * NC + cid
        il = lax.iota(jnp.int32, 16)
        r0 = sid * RPT

        pltpu.sync_copy(zeros_hbm.at[pl.ds(r0, RPT)], acc.at[pl.ds(r0, RPT)])
        pltpu.sync_copy(mx_hbm, mxbuf)

        def mred(i, mv):
            return jnp.maximum(mv, mxbuf[i])

        mv = lax.fori_loop(0, NW, mred, jnp.full((16,), -1e30, jnp.float32))
        gmax = jnp.max(mv)
        plsc.subcore_barrier()

        def chunk(ci, _):
            e0 = wid * EPW + ci * T
            pltpu.sync_copy(dst_hbm.at[pl.ds(e0, T)], dst_i)
            pltpu.sync_copy(src_hbm.at[pl.ds(e0, T)], src_i)
            cv = pltpu.async_copy(v_hbm.at[src_i], vbuf, sem0)
            pltpu.sync_copy(alpha_hbm.at[wid * NCH + ci], exbuf)
            cv.wait()

            def edge(e, _):
                a = exbuf[e]
                valid = jnp.logical_and(il < H, e0 + e < E_real)
                ex = jnp.where(valid, jnp.exp(a - gmax), 0.0)
                for h in range(H):
                    w = jnp.sum(jnp.where(il == h, ex, 0.0))
                    wvbuf[e, pl.ds(h * C, 16)] = vbuf[e, pl.ds(h * C, 16)] * w
                wvbuf[e, pl.ds(HID, 16)] = ex
                return 0

            lax.fori_loop(0, T, edge, 0)
            pltpu.sync_copy(wvbuf, acc.at[dst_i], add=True)
            return 0

        lax.fori_loop(0, NCH, chunk, 0)
        plsc.subcore_barrier()
        pltpu.sync_copy(acc.at[pl.ds(r0, RPT)], out_hbm.at[cid, pl.ds(r0, RPT)])

    return pass1, pass2


def _fin_body(az_ref, ah_ref, sz_ref, sh_ref, o_ref):
    az = az_ref[0] + az_ref[1]
    ah = ah_ref[0] + ah_ref[1]
    blk = az.shape[0]

    def norm(a):
        num = a[:, 0:HID]
        den = a[:, HID:HID + H]
        dexp = jnp.concatenate(
            [jnp.broadcast_to(den[:, h:h + 1], (blk, C)) for h in range(H)],
            axis=1)
        return num / (dexp + 1e-16)

    z = jax.nn.sigmoid(norm(az) + sz_ref[...])
    ht = jnp.tanh(norm(ah) + sh_ref[...])
    o_ref[...] = (1.0 - z) * ht


@functools.lru_cache(maxsize=None)
def _make_finalize(N):
    BLK = 2000
    grid = N // BLK
    return pl.pallas_call(
        _fin_body,
        grid=(grid,),
        in_specs=[
            pl.BlockSpec((NC, BLK, ACC_W), lambda i: (0, i, 0)),
            pl.BlockSpec((NC, BLK, ACC_W), lambda i: (0, i, 0)),
            pl.BlockSpec((BLK, HID), lambda i: (i, 0)),
            pl.BlockSpec((BLK, HID), lambda i: (i, 0)),
        ],
        out_specs=pl.BlockSpec((BLK, HID), lambda i: (i, 0)),
        out_shape=jax.ShapeDtypeStruct((N, HID), jnp.float32),
    )


def kernel(x, edge_index, params):
    N, in_ch = x.shape
    E = edge_index.shape[1]
    src = edge_index[0].astype(jnp.int32)
    dst = edge_index[1].astype(jnp.int32)

    chunk_stride = NW * T
    E_pad = ((E + chunk_stride - 1) // chunk_stride) * chunk_stride
    pad = E_pad - E
    srcp = jnp.concatenate([src, jnp.zeros((pad,), jnp.int32)])
    dstp = jnp.concatenate([dst, jnp.zeros((pad,), jnp.int32)])

    def wb(p):
        W = jnp.concatenate(
            [p['Wq'][:in_ch], p['Wk'][:in_ch], p['Wv'][:in_ch], p['Ws'][:in_ch]],
            axis=1)
        b = jnp.concatenate([p['bq'], p['bk'], p['bv'], p['bs']])[None, :]
        return W, b

    project = _make_project(N)
    Wz, bz = wb(params['z'])
    Wh, bh = wb(params['h'])
    qz, kz, vz, sz = project(x, Wz, bz)
    qh, kh, vh, sh = project(x, Wh, bh)

    pass1, pass2 = _make_sc(N, E_pad, E)
    alpha_z, mx_z = pass1(dstp, srcp, qz, kz)
    alpha_h, mx_h = pass1(dstp, srcp, qh, kh)
    zeros = jnp.zeros((N, ACC_W), jnp.float32)
    acc_z = pass2(dstp, srcp, vz, alpha_z, mx_z, zeros)
    acc_h = pass2(dstp, srcp, vh, alpha_h, mx_h, zeros)

    finalize = _make_finalize(N)
    return finalize(acc_z, acc_h, sz, sh)
